# baseline (speedup 1.0000x reference)
import jax
import jax.numpy as jnp
from jax.experimental import pallas as pl


def _expert0_matmul_kernel(x_ref, w_ref, o_ref):
    o_ref[...] = jax.lax.dot_general(
        x_ref[...].astype(jnp.bfloat16),
        w_ref[...].astype(jnp.bfloat16),
        dimension_numbers=(((1,), (1,)), ((), ())),
        preferred_element_type=jnp.float32,
    )


def kernel(inputs, router_w, expert_ws):
    del router_w
    w0 = expert_ws[0]
    m, k = inputs.shape
    n = w0.shape[0]
    bm = 2048
    bn = 512
    return pl.pallas_call(
        _expert0_matmul_kernel,
        grid=(m // bm, n // bn),
        in_specs=[
            pl.BlockSpec((bm, k), lambda i, j: (i, 0)),
            pl.BlockSpec((bn, k), lambda i, j: (j, 0)),
        ],
        out_specs=pl.BlockSpec((bm, bn), lambda i, j: (i, j)),
        out_shape=jax.ShapeDtypeStruct((m, n), inputs.dtype),
    )(inputs, w0)


# speedup vs baseline: 1.3165x; 1.3165x over previous
import jax
import jax.numpy as jnp
from jax.experimental import pallas as pl


def _expert0_matmul_kernel(x_ref, w_ref, o_ref):
    o_ref[...] = jax.lax.dot_general(
        x_ref[...],
        w_ref[...],
        dimension_numbers=(((1,), (1,)), ((), ())),
        preferred_element_type=jnp.float32,
    )


def kernel(inputs, router_w, expert_ws):
    del router_w
    w0 = expert_ws[0]
    m, k = inputs.shape
    n = w0.shape[0]
    bm = 2048
    return pl.pallas_call(
        _expert0_matmul_kernel,
        grid=(m // bm,),
        in_specs=[
            pl.BlockSpec((bm, k), lambda i: (i, 0)),
            pl.BlockSpec((n, k), lambda i: (0, 0)),
        ],
        out_specs=pl.BlockSpec((bm, n), lambda i: (i, 0)),
        out_shape=jax.ShapeDtypeStruct((m, n), inputs.dtype),
    )(inputs, w0)


# speedup vs baseline: 1.3544x; 1.0288x over previous
import jax
import jax.numpy as jnp
from jax.experimental import pallas as pl
from jax.experimental.pallas import tpu as pltpu

_CH = 1024
_NIN = 4
_NOUT = 3


def _pipelined_matmul_kernel(x_hbm, w_hbm, o_hbm, xb, wb, ob,
                             in_sems, out_sems, w_sem):
    n_chunks = x_hbm.shape[0] // _CH

    def in_copy(chunk, buf):
        return pltpu.make_async_copy(
            x_hbm.at[pl.ds(chunk * _CH, _CH)], xb.at[buf], in_sems.at[buf])

    def out_copy(chunk, buf):
        return pltpu.make_async_copy(
            ob.at[buf], o_hbm.at[pl.ds(chunk * _CH, _CH)], out_sems.at[buf])

    w_copy = pltpu.make_async_copy(w_hbm, wb, w_sem)
    w_copy.start()
    for b in range(min(_NIN, n_chunks)):
        in_copy(b, b).start()
    w_copy.wait()

    for i in range(n_chunks):
        ib = i % _NIN
        outb = i % _NOUT
        in_copy(i, ib).wait()
        if i >= _NOUT:
            out_copy(i - _NOUT, outb).wait()
        ob[outb] = jax.lax.dot_general(
            xb[ib].astype(jnp.bfloat16),
            wb[...].astype(jnp.bfloat16),
            dimension_numbers=(((1,), (1,)), ((), ())),
            preferred_element_type=jnp.float32,
        )
        out_copy(i, outb).start()
        nxt = i + _NIN
        if nxt < n_chunks:
            in_copy(nxt, ib).start()

    for i in range(max(0, n_chunks - _NOUT), n_chunks):
        out_copy(i, i % _NOUT).wait()


def kernel(inputs, router_w, expert_ws):
    del router_w
    w0 = expert_ws[0]
    m, k = inputs.shape
    n = w0.shape[0]
    return pl.pallas_call(
        _pipelined_matmul_kernel,
        in_specs=[
            pl.BlockSpec(memory_space=pltpu.MemorySpace.HBM),
            pl.BlockSpec(memory_space=pltpu.MemorySpace.HBM),
        ],
        out_specs=pl.BlockSpec(memory_space=pltpu.MemorySpace.HBM),
        out_shape=jax.ShapeDtypeStruct((m, n), inputs.dtype),
        scratch_shapes=[
            pltpu.VMEM((_NIN, _CH, k), jnp.float32),
            pltpu.VMEM((n, k), jnp.float32),
            pltpu.VMEM((_NOUT, _CH, n), jnp.float32),
            pltpu.SemaphoreType.DMA((_NIN,)),
            pltpu.SemaphoreType.DMA((_NOUT,)),
            pltpu.SemaphoreType.DMA,
        ],
    )(inputs, w0)
